# baseline (speedup 1.0000x reference)
import functools

import jax
import jax.numpy as jnp
import numpy as np
from jax import lax
from jax.experimental import pallas as pl
from jax.experimental.pallas import tpu as pltpu
from jax.experimental.pallas import tpu_sc as plsc

DP = 768
NP = 10240
NW = 32
NPW = NP // NW
SG = 8
NG = NPW // SG
KB = 32
NCH = DP // 16
FBIG = 3.0e38


def _pad2(a, r, c):
    return jnp.pad(a, ((0, r - a.shape[0]), (0, c - a.shape[1])))


def _pad1(a, r):
    return jnp.pad(a, ((0, r - a.shape[0]),))


def _mm(x, w, b, act=None, bm=1024, bk=768):
    M, K = x.shape
    Nw = w.shape[1]
    nk = K // bk

    def body(x_ref, w_ref, b_ref, o_ref, acc_ref):
        k = pl.program_id(1)

        @pl.when(k == 0)
        def _():
            acc_ref[...] = jnp.zeros_like(acc_ref)

        acc_ref[...] += jnp.dot(x_ref[...], w_ref[...],
                                precision=jax.lax.Precision.HIGHEST,
                                preferred_element_type=jnp.float32)

        @pl.when(k == nk - 1)
        def _():
            r = acc_ref[...] + b_ref[...]
            if act == "relu":
                r = jnp.maximum(r, 0.0)
            o_ref[...] = r

    return pl.pallas_call(
        body,
        grid=(M // bm, nk),
        in_specs=[
            pl.BlockSpec((bm, bk), lambda i, k: (i, k)),
            pl.BlockSpec((bk, Nw), lambda i, k: (k, 0)),
            pl.BlockSpec((1, Nw), lambda i, k: (0, 0)),
        ],
        out_specs=pl.BlockSpec((bm, Nw), lambda i, k: (i, 0)),
        out_shape=jax.ShapeDtypeStruct((M, Nw), jnp.float32),
        scratch_shapes=[pltpu.VMEM((bm, Nw), jnp.float32)],
    )(x, w, b.reshape(1, -1))


def _post(h, s, q, mn, mx, cnt, wstack, b, bm=512):
    M, Dp = h.shape

    def body(h_ref, s_ref, q_ref, mn_ref, mx_ref, c_ref, w_ref, b_ref, o_ref):
        c = c_ref[...]
        has = c > 0.0
        cc = jnp.maximum(c, 1.0)
        sv = s_ref[...]
        mean = sv / cc
        std = jnp.sqrt(jnp.maximum(q_ref[...] / cc - mean * mean, 0.0) + 1e-5)
        mnv = jnp.where(has, mn_ref[...], 0.0)
        mxv = jnp.where(has, mx_ref[...], 0.0)
        dot = functools.partial(jnp.dot,
                                precision=jax.lax.Precision.HIGHEST,
                                preferred_element_type=jnp.float32)
        acc = dot(h_ref[...], w_ref[0])
        acc += dot(sv, w_ref[1])
        acc += dot(mean, w_ref[2])
        acc += dot(mnv, w_ref[3])
        acc += dot(mxv, w_ref[4])
        acc += dot(std, w_ref[5])
        o_ref[...] = acc + b_ref[...]

    blk = lambda i: (i, 0)
    return pl.pallas_call(
        body,
        grid=(M // bm,),
        in_specs=[
            pl.BlockSpec((bm, Dp), blk),
            pl.BlockSpec((bm, Dp), blk),
            pl.BlockSpec((bm, Dp), blk),
            pl.BlockSpec((bm, Dp), blk),
            pl.BlockSpec((bm, Dp), blk),
            pl.BlockSpec((bm, 1), blk),
            pl.BlockSpec((6, Dp, Dp), lambda i: (0, 0, 0)),
            pl.BlockSpec((1, Dp), lambda i: (0, 0)),
        ],
        out_specs=pl.BlockSpec((bm, Dp), blk),
        out_shape=jax.ShapeDtypeStruct((M, Dp), jnp.float32),
    )(h, s, q, mn, mx, cnt, wstack, b.reshape(1, -1))


def _bn_stats(h, nreal, bm=2048):
    M, Dp = h.shape

    def body(h_ref, o_ref):
        i = pl.program_id(0)

        @pl.when(i == 0)
        def _():
            o_ref[...] = jnp.zeros_like(o_ref)

        hv = h_ref[...]
        row = jax.lax.broadcasted_iota(jnp.int32, (bm, 1), 0) + i * bm
        msk = (row < nreal).astype(jnp.float32)
        hm = hv * msk
        o_ref[0:1, :] += jnp.sum(hm, axis=0, keepdims=True)
        o_ref[1:2, :] += jnp.sum(hm * hv, axis=0, keepdims=True)

    return pl.pallas_call(
        body,
        grid=(M // bm,),
        in_specs=[pl.BlockSpec((bm, Dp), lambda i: (i, 0))],
        out_specs=pl.BlockSpec((8, Dp), lambda i: (0, 0)),
        out_shape=jax.ShapeDtypeStruct((8, Dp), jnp.float32),
    )(h)


def _bn_apply_relu(h, scale, shift, bm=2048):
    M, Dp = h.shape

    def body(h_ref, sc_ref, sh_ref, o_ref):
        o_ref[...] = jnp.maximum(h_ref[...] * sc_ref[...] + sh_ref[...], 0.0)

    return pl.pallas_call(
        body,
        grid=(M // bm,),
        in_specs=[
            pl.BlockSpec((bm, Dp), lambda i: (i, 0)),
            pl.BlockSpec((1, Dp), lambda i: (0, 0)),
            pl.BlockSpec((1, Dp), lambda i: (0, 0)),
        ],
        out_specs=pl.BlockSpec((bm, Dp), lambda i: (i, 0)),
        out_shape=jax.ShapeDtypeStruct((M, Dp), jnp.float32),
    )(h, scale.reshape(1, -1), shift.reshape(1, -1))


def _vext(ref, idx):
    return ref[pl.ds(idx, 16)][0]


def _sc_segment(A, B, T, src_s, attr_s, dst_s, offw):
    mesh = plsc.VectorSubcoreMesh(core_axis_name="c", subcore_axis_name="s")
    row_t = jax.ShapeDtypeStruct((NP, DP), jnp.float32)

    @functools.partial(
        pl.kernel,
        out_type=[row_t, row_t, row_t, row_t,
                  jax.ShapeDtypeStruct((NP, 16), jnp.float32)],
        mesh=mesh,
        scratch_types=[
            pltpu.VMEM((KB,), jnp.int32),
            pltpu.VMEM((KB + 16,), jnp.int32),
            pltpu.VMEM((KB + 16,), jnp.int32),
            pltpu.VMEM((KB, DP), jnp.float32),
            pltpu.VMEM((32, DP), jnp.float32),
            pltpu.VMEM((2 * SG, DP), jnp.float32),
            pltpu.VMEM((2 * SG, DP), jnp.float32),
            pltpu.VMEM((2 * SG, DP), jnp.float32),
            pltpu.VMEM((2 * SG, DP), jnp.float32),
            pltpu.VMEM((2 * SG, DP), jnp.float32),
            pltpu.VMEM((2 * SG, 16), jnp.float32),
            pltpu.VMEM((64,), jnp.int32),
            pltpu.SMEM((1,), jnp.int32),
            pltpu.SemaphoreType.DMA,
            pltpu.SemaphoreType.DMA,
            pltpu.SemaphoreType.DMA,
        ],
    )
    def seg(A_h, B_h, T_h, src_h, attr_h, dst_h, offw_h,
            S_h, Q_h, MN_h, MX_h, C_h,
            srcv, attrv, dstv, Bbuf, Tbuf, Agrp, stS, stQ, stMN, stMX,
            cntst, offv, cgr, semf0, semf1, semg):
        w = lax.axis_index("s") * 2 + lax.axis_index("c")
        n0 = w * NPW
        pltpu.sync_copy(T_h, Tbuf)
        pltpu.sync_copy(offw_h, offv)
        e0 = _vext(offv, w)
        e1 = _vext(offv, w + 1)
        e0a = (e0 // KB) * KB

        def zero_fill(half):
            base = half * SG

            def zrow(r, _):
                def zch(c, _):
                    sl = pl.ds(c * 16, 16)
                    z = jnp.zeros((16,), jnp.float32)
                    stS[base + r, sl] = z
                    stQ[base + r, sl] = z
                    stMN[base + r, sl] = jnp.full((16,), FBIG, jnp.float32)
                    stMX[base + r, sl] = jnp.full((16,), -FBIG, jnp.float32)
                    return 0

                lax.fori_loop(0, NCH, zch, 0)
                cntst[base + r, :] = jnp.zeros((16,), jnp.float32)
                return 0

            lax.fori_loop(0, SG, zrow, 0)

        def wait_flush(sem):
            gslab = pl.ds(0, SG)
            for ref in (S_h, Q_h, MN_h, MX_h):
                pltpu.make_async_copy(stS.at[gslab], ref.at[gslab], sem).wait()
            pltpu.make_async_copy(cntst.at[gslab], C_h.at[gslab], sem).wait()

        def issue_flush(g, half, sem):
            hslab = pl.ds(half * SG, SG)
            oslab = pl.ds(n0 + g * SG, SG)
            pltpu.async_copy(stS.at[hslab], S_h.at[oslab], sem)
            pltpu.async_copy(stQ.at[hslab], Q_h.at[oslab], sem)
            pltpu.async_copy(stMN.at[hslab], MN_h.at[oslab], sem)
            pltpu.async_copy(stMX.at[hslab], MX_h.at[oslab], sem)
            pltpu.async_copy(cntst.at[hslab], C_h.at[oslab], sem)

        def enter_group(g):
            half = g % 2

            def prep(sem):
                @pl.when(g >= 2)
                def _():
                    wait_flush(sem)

                zero_fill(half)

            lax.cond(half == 0, lambda: prep(semf0), lambda: prep(semf1))

            @pl.when(g < NG)
            def _():
                pltpu.sync_copy(
                    A_h.at[pl.ds(n0 + g * SG, SG)],
                    Agrp.at[pl.ds(half * SG, SG)])

        def close_group(g):
            half = g % 2
            lax.cond(half == 0,
                     lambda: issue_flush(g, 0, semf0),
                     lambda: issue_flush(g, 1, semf1))

        cgr[0] = jnp.int32(0)
        enter_group(0)

        def advance_to(g_t):
            cur_g = cgr[0]

            @pl.when(g_t > cur_g)
            def _():
                @pl.loop(0, NG)
                def _(k):
                    g = cur_g + k

                    @pl.when(g < g_t)
                    def _():
                        close_group(g)
                        enter_group(g + 1)

                cgr[0] = g_t

        nb = (e1 - e0a + KB - 1) // KB
        nbmax = (src_s.shape[0] // KB) + 1

        @pl.loop(0, nbmax)
        def _(bi):
            @pl.when(bi < nb)
            def _():
                bbase = e0a + bi * KB
                pltpu.sync_copy(src_h.at[pl.ds(bbase, KB)], srcv)
                pltpu.sync_copy(attr_h.at[pl.ds(bbase, KB)],
                                attrv.at[pl.ds(0, KB)])
                pltpu.sync_copy(dst_h.at[pl.ds(bbase, KB)],
                                dstv.at[pl.ds(0, KB)])
                pltpu.async_copy(B_h.at[srcv], Bbuf, semg).wait()
                i0 = jnp.maximum(e0 - bbase, 0)
                i1 = jnp.minimum(KB, e1 - bbase)

                @pl.loop(0, KB)
                def _(i):
                    @pl.when(jnp.logical_and(i >= i0, i < i1))
                    def _():
                        n = _vext(dstv, i)
                        at = _vext(attrv, i)
                        g_t = (n - n0) // SG
                        advance_to(g_t)
                        row = (g_t % 2) * SG + (n - n0) % SG
                        cntst[row, :] += jnp.full((16,), 1.0, jnp.float32)

                        @pl.loop(0, NCH, unroll=True)
                        def _(c):
                            sl = pl.ds(c * 16, 16)
                            m = Agrp[row, sl] + Bbuf[i, sl] + Tbuf[at, sl]
                            stS[row, sl] += m
                            stQ[row, sl] += m * m
                            stMN[row, sl] = jnp.minimum(stMN[row, sl], m)
                            stMX[row, sl] = jnp.maximum(stMX[row, sl], m)

        advance_to(jnp.int32(NG))
        wait_flush(semf1)

    return seg(A, B, T, src_s, attr_s, dst_s, offw)


def kernel(x, edge_index, edge_attr, params):
    n, d = x.shape
    src = edge_index[0]
    dst = edge_index[1]

    pre = params["pre_mlp"]
    W_pre_mlp = _pad2(pre["W"], DP, DP)
    b_pre_mlp = _pad1(pre["b"], DP)

    xp = _pad2(x, NP, DP)
    h = _mm(xp, W_pre_mlp, b_pre_mlp, act="relu")

    e_table = params["edge_emb"]
    wagg = params["agg_weights"]

    order = jnp.argsort(dst)
    dst_s = dst[order].astype(jnp.int32)
    src_s = src[order].astype(jnp.int32)
    attr_s = edge_attr[order].astype(jnp.int32)
    offw = jnp.searchsorted(dst_s, jnp.arange(NW + 1, dtype=jnp.int32) * NPW,
                            side="left").astype(jnp.int32)
    offw = _pad1(offw, 64)
    dst_s = _pad1(dst_s, dst_s.shape[0] + KB)
    src_s = _pad1(src_s, src_s.shape[0] + KB)
    attr_s = _pad1(attr_s, attr_s.shape[0] + KB)

    for cp in params["convs"]:
        Wp = cp["pre_nn"]["W"]
        bp = cp["pre_nn"]["b"]
        W1 = _pad2(Wp[:d], DP, DP)
        W2 = _pad2(Wp[d:2 * d], DP, DP)
        W3 = Wp[2 * d:]
        e_enc = e_table @ cp["edge_enc"]["W"] + cp["edge_enc"]["b"]
        T = _pad2(e_enc @ W3 + bp, 32, DP)

        AB = _mm(h, jnp.concatenate([W1, W2], axis=1),
                 jnp.zeros((2 * DP,), jnp.float32))
        A = AB[:, :DP]
        B = AB[:, DP:]

        s, q, mn, mx, cntw = _sc_segment(A, B, T, src_s, attr_s, dst_s, offw)
        cntp = cntw[:, :1]

        Wpost = cp["post_nn"]["W"]
        blocks = [Wpost[i * d:(i + 1) * d] for i in range(6)]
        scaled = [blocks[0]] + [wagg[i] * blocks[i + 1] for i in range(5)]
        wstack = jnp.stack([_pad2(wb, DP, DP) for wb in scaled])
        hpost = _post(h, s, q, mn, mx, cntp, wstack,
                      _pad1(cp["post_nn"]["b"], DP))

        h = _mm(hpost, _pad2(cp["lin"]["W"], DP, DP),
                _pad1(cp["lin"]["b"], DP))

        stats = _bn_stats(h, n)
        mu = stats[0] / n
        var = stats[1] / n - mu * mu
        gamma = _pad1(cp["bn_gamma"], DP)
        beta = _pad1(cp["bn_beta"], DP)
        scale = gamma * jax.lax.rsqrt(var + 1e-5)
        shift = beta - mu * scale
        h = _bn_apply_relu(h, scale, shift)

    f = _mm(h, _pad2(params["fp1"]["W"], DP, 384),
            _pad1(params["fp1"]["b"], 384), act="relu")
    out = _mm(f, _pad2(params["fp2"]["W"], 384, 128),
              _pad1(params["fp2"]["b"], 128), bk=384)
    return out[:n, :3]


# speedup vs baseline: 1.1187x; 1.1187x over previous
import functools

import jax
import jax.numpy as jnp
import numpy as np
from jax import lax
from jax.experimental import pallas as pl
from jax.experimental.pallas import tpu as pltpu
from jax.experimental.pallas import tpu_sc as plsc

DP = 768
NP = 10240
NW = 32
NPW = NP // NW
SG = 8
NG = NPW // SG
KB = 32
NCH = DP // 16
FBIG = 3.0e38


def _pad2(a, r, c):
    return jnp.pad(a, ((0, r - a.shape[0]), (0, c - a.shape[1])))


def _pad1(a, r):
    return jnp.pad(a, ((0, r - a.shape[0]),))


def _mm(x, w, b, act=None, bm=1024, bk=768):
    M, K = x.shape
    Nw = w.shape[1]
    nk = K // bk

    def body(x_ref, w_ref, b_ref, o_ref, acc_ref):
        k = pl.program_id(1)

        @pl.when(k == 0)
        def _():
            acc_ref[...] = jnp.zeros_like(acc_ref)

        acc_ref[...] += jnp.dot(x_ref[...], w_ref[...],
                                precision=jax.lax.Precision.HIGHEST,
                                preferred_element_type=jnp.float32)

        @pl.when(k == nk - 1)
        def _():
            r = acc_ref[...] + b_ref[...]
            if act == "relu":
                r = jnp.maximum(r, 0.0)
            o_ref[...] = r

    return pl.pallas_call(
        body,
        grid=(M // bm, nk),
        in_specs=[
            pl.BlockSpec((bm, bk), lambda i, k: (i, k)),
            pl.BlockSpec((bk, Nw), lambda i, k: (k, 0)),
            pl.BlockSpec((1, Nw), lambda i, k: (0, 0)),
        ],
        out_specs=pl.BlockSpec((bm, Nw), lambda i, k: (i, 0)),
        out_shape=jax.ShapeDtypeStruct((M, Nw), jnp.float32),
        scratch_shapes=[pltpu.VMEM((bm, Nw), jnp.float32)],
    )(x, w, b.reshape(1, -1))


def _post(h, s, q, mn, mx, cnt, wstack, b, bm=512):
    M, Dp = h.shape

    def body(h_ref, s_ref, q_ref, mn_ref, mx_ref, c_ref, w_ref, b_ref, o_ref):
        c = c_ref[...]
        has = c > 0.0
        cc = jnp.maximum(c, 1.0)
        sv = s_ref[...]
        mean = sv / cc
        std = jnp.sqrt(jnp.maximum(q_ref[...] / cc - mean * mean, 0.0) + 1e-5)
        mnv = jnp.where(has, mn_ref[...], 0.0)
        mxv = jnp.where(has, mx_ref[...], 0.0)
        dot = functools.partial(jnp.dot,
                                precision=jax.lax.Precision.HIGHEST,
                                preferred_element_type=jnp.float32)
        acc = dot(h_ref[...], w_ref[0])
        acc += dot(sv, w_ref[1])
        acc += dot(mean, w_ref[2])
        acc += dot(mnv, w_ref[3])
        acc += dot(mxv, w_ref[4])
        acc += dot(std, w_ref[5])
        o_ref[...] = acc + b_ref[...]

    blk = lambda i: (i, 0)
    return pl.pallas_call(
        body,
        grid=(M // bm,),
        in_specs=[
            pl.BlockSpec((bm, Dp), blk),
            pl.BlockSpec((bm, Dp), blk),
            pl.BlockSpec((bm, Dp), blk),
            pl.BlockSpec((bm, Dp), blk),
            pl.BlockSpec((bm, Dp), blk),
            pl.BlockSpec((bm, 1), blk),
            pl.BlockSpec((6, Dp, Dp), lambda i: (0, 0, 0)),
            pl.BlockSpec((1, Dp), lambda i: (0, 0)),
        ],
        out_specs=pl.BlockSpec((bm, Dp), blk),
        out_shape=jax.ShapeDtypeStruct((M, Dp), jnp.float32),
    )(h, s, q, mn, mx, cnt, wstack, b.reshape(1, -1))


def _bn_stats(h, nreal, bm=2048):
    M, Dp = h.shape

    def body(h_ref, o_ref):
        i = pl.program_id(0)

        @pl.when(i == 0)
        def _():
            o_ref[...] = jnp.zeros_like(o_ref)

        hv = h_ref[...]
        row = jax.lax.broadcasted_iota(jnp.int32, (bm, 1), 0) + i * bm
        msk = (row < nreal).astype(jnp.float32)
        hm = hv * msk
        o_ref[0:1, :] += jnp.sum(hm, axis=0, keepdims=True)
        o_ref[1:2, :] += jnp.sum(hm * hv, axis=0, keepdims=True)

    return pl.pallas_call(
        body,
        grid=(M // bm,),
        in_specs=[pl.BlockSpec((bm, Dp), lambda i: (i, 0))],
        out_specs=pl.BlockSpec((8, Dp), lambda i: (0, 0)),
        out_shape=jax.ShapeDtypeStruct((8, Dp), jnp.float32),
    )(h)


def _bn_apply_relu(h, scale, shift, bm=2048):
    M, Dp = h.shape

    def body(h_ref, sc_ref, sh_ref, o_ref):
        o_ref[...] = jnp.maximum(h_ref[...] * sc_ref[...] + sh_ref[...], 0.0)

    return pl.pallas_call(
        body,
        grid=(M // bm,),
        in_specs=[
            pl.BlockSpec((bm, Dp), lambda i: (i, 0)),
            pl.BlockSpec((1, Dp), lambda i: (0, 0)),
            pl.BlockSpec((1, Dp), lambda i: (0, 0)),
        ],
        out_specs=pl.BlockSpec((bm, Dp), lambda i: (i, 0)),
        out_shape=jax.ShapeDtypeStruct((M, Dp), jnp.float32),
    )(h, scale.reshape(1, -1), shift.reshape(1, -1))


def _vext(ref, idx):
    return ref[pl.ds(idx, 16)][0]


def _sc_segment(A, B, T, src_s, attr_s, dst_s, offw):
    mesh = plsc.VectorSubcoreMesh(core_axis_name="c", subcore_axis_name="s")
    row_t = jax.ShapeDtypeStruct((NP, DP), jnp.float32)

    @functools.partial(
        pl.kernel,
        out_type=[row_t, row_t, row_t, row_t,
                  jax.ShapeDtypeStruct((NP, 16), jnp.float32)],
        mesh=mesh,
        scratch_types=[
            pltpu.VMEM((KB,), jnp.int32),
            pltpu.VMEM((KB + 16,), jnp.int32),
            pltpu.VMEM((KB + 16,), jnp.int32),
            pltpu.VMEM((KB, DP), jnp.float32),
            pltpu.VMEM((32, DP), jnp.float32),
            pltpu.VMEM((2 * SG, DP), jnp.float32),
            pltpu.VMEM((2 * SG, DP), jnp.float32),
            pltpu.VMEM((2 * SG, DP), jnp.float32),
            pltpu.VMEM((2 * SG, DP), jnp.float32),
            pltpu.VMEM((2 * SG, DP), jnp.float32),
            pltpu.VMEM((2 * SG, 16), jnp.float32),
            pltpu.VMEM((64,), jnp.int32),
            pltpu.SMEM((1,), jnp.int32),
            pltpu.SemaphoreType.DMA,
            pltpu.SemaphoreType.DMA,
            pltpu.SemaphoreType.DMA,
        ],
    )
    def seg(A_h, B_h, T_h, src_h, attr_h, dst_h, offw_h,
            S_h, Q_h, MN_h, MX_h, C_h,
            srcv, attrv, dstv, Bbuf, Tbuf, Agrp, stS, stQ, stMN, stMX,
            cntst, offv, cgr, semf0, semf1, semg):
        w = lax.axis_index("s") * 2 + lax.axis_index("c")
        n0 = w * NPW
        pltpu.sync_copy(T_h, Tbuf)
        pltpu.sync_copy(offw_h, offv)
        e0 = _vext(offv, w)
        e1 = _vext(offv, w + 1)
        e0a = (e0 // KB) * KB

        def zero_fill(half):
            base = half * SG

            def zrow(r, _):
                def zch(c, _):
                    sl = pl.ds(c * 16, 16)
                    z = jnp.zeros((16,), jnp.float32)
                    stS[base + r, sl] = z
                    stQ[base + r, sl] = z
                    stMN[base + r, sl] = jnp.full((16,), FBIG, jnp.float32)
                    stMX[base + r, sl] = jnp.full((16,), -FBIG, jnp.float32)
                    return 0

                lax.fori_loop(0, NCH, zch, 0)
                cntst[base + r, :] = jnp.zeros((16,), jnp.float32)
                return 0

            lax.fori_loop(0, SG, zrow, 0)

        def wait_flush(sem):
            gslab = pl.ds(0, SG)
            for ref in (S_h, Q_h, MN_h, MX_h):
                pltpu.make_async_copy(stS.at[gslab], ref.at[gslab], sem).wait()
            pltpu.make_async_copy(cntst.at[gslab], C_h.at[gslab], sem).wait()

        def issue_flush(g, half, sem):
            hslab = pl.ds(half * SG, SG)
            oslab = pl.ds(n0 + g * SG, SG)
            pltpu.async_copy(stS.at[hslab], S_h.at[oslab], sem)
            pltpu.async_copy(stQ.at[hslab], Q_h.at[oslab], sem)
            pltpu.async_copy(stMN.at[hslab], MN_h.at[oslab], sem)
            pltpu.async_copy(stMX.at[hslab], MX_h.at[oslab], sem)
            pltpu.async_copy(cntst.at[hslab], C_h.at[oslab], sem)

        def enter_group(g):
            half = g % 2

            def prep(sem):
                @pl.when(g >= 2)
                def _():
                    wait_flush(sem)

                zero_fill(half)

            lax.cond(half == 0, lambda: prep(semf0), lambda: prep(semf1))

            @pl.when(g < NG)
            def _():
                pltpu.sync_copy(
                    A_h.at[pl.ds(n0 + g * SG, SG)],
                    Agrp.at[pl.ds(half * SG, SG)])

        def close_group(g):
            half = g % 2
            lax.cond(half == 0,
                     lambda: issue_flush(g, 0, semf0),
                     lambda: issue_flush(g, 1, semf1))

        cgr[0] = jnp.int32(0)
        enter_group(0)

        def advance_to(g_t):
            cur_g = cgr[0]

            @pl.when(g_t > cur_g)
            def _():
                @pl.loop(0, NG)
                def _(k):
                    g = cur_g + k

                    @pl.when(g < g_t)
                    def _():
                        close_group(g)
                        enter_group(g + 1)

                cgr[0] = g_t

        nb = (e1 - e0a + KB - 1) // KB
        nbmax = (src_s.shape[0] // KB) + 1

        @pl.loop(0, nbmax)
        def _(bi):
            @pl.when(bi < nb)
            def _():
                bbase = e0a + bi * KB
                pltpu.sync_copy(src_h.at[pl.ds(bbase, KB)], srcv)
                pltpu.sync_copy(attr_h.at[pl.ds(bbase, KB)],
                                attrv.at[pl.ds(0, KB)])
                pltpu.sync_copy(dst_h.at[pl.ds(bbase, KB)],
                                dstv.at[pl.ds(0, KB)])
                pltpu.async_copy(B_h.at[srcv], Bbuf, semg).wait()
                i0 = jnp.maximum(e0 - bbase, 0)
                i1 = jnp.minimum(KB, e1 - bbase)

                @pl.loop(0, KB)
                def _(i):
                    @pl.when(jnp.logical_and(i >= i0, i < i1))
                    def _():
                        n = _vext(dstv, i)
                        at = _vext(attrv, i)
                        g_t = (n - n0) // SG
                        advance_to(g_t)
                        row = (g_t % 2) * SG + (n - n0) % SG
                        cntst[row, :] += jnp.full((16,), 1.0, jnp.float32)

                        @pl.loop(0, NCH)
                        def _(c):
                            sl = pl.ds(c * 16, 16)
                            m = Agrp[row, sl] + Bbuf[i, sl] + Tbuf[at, sl]
                            stS[row, sl] += m
                            stQ[row, sl] += m * m
                            stMN[row, sl] = jnp.minimum(stMN[row, sl], m)
                            stMX[row, sl] = jnp.maximum(stMX[row, sl], m)

        advance_to(jnp.int32(NG))
        wait_flush(semf1)

    return seg(A, B, T, src_s, attr_s, dst_s, offw)


def kernel(x, edge_index, edge_attr, params):
    n, d = x.shape
    src = edge_index[0]
    dst = edge_index[1]

    pre = params["pre_mlp"]
    W_pre_mlp = _pad2(pre["W"], DP, DP)
    b_pre_mlp = _pad1(pre["b"], DP)

    xp = _pad2(x, NP, DP)
    h = _mm(xp, W_pre_mlp, b_pre_mlp, act="relu")

    e_table = params["edge_emb"]
    wagg = params["agg_weights"]

    order = jnp.argsort(dst)
    dst_s = dst[order].astype(jnp.int32)
    src_s = src[order].astype(jnp.int32)
    attr_s = edge_attr[order].astype(jnp.int32)
    offw = jnp.searchsorted(dst_s, jnp.arange(NW + 1, dtype=jnp.int32) * NPW,
                            side="left").astype(jnp.int32)
    offw = _pad1(offw, 64)
    dst_s = _pad1(dst_s, dst_s.shape[0] + KB)
    src_s = _pad1(src_s, src_s.shape[0] + KB)
    attr_s = _pad1(attr_s, attr_s.shape[0] + KB)

    for cp in params["convs"]:
        Wp = cp["pre_nn"]["W"]
        bp = cp["pre_nn"]["b"]
        W1 = _pad2(Wp[:d], DP, DP)
        W2 = _pad2(Wp[d:2 * d], DP, DP)
        W3 = Wp[2 * d:]
        hp = jax.lax.Precision.HIGHEST
        e_enc = jnp.dot(e_table, cp["edge_enc"]["W"],
                        precision=hp) + cp["edge_enc"]["b"]
        T = _pad2(jnp.dot(e_enc, W3, precision=hp) + bp, 32, DP)

        AB = _mm(h, jnp.concatenate([W1, W2], axis=1),
                 jnp.zeros((2 * DP,), jnp.float32))
        A = AB[:, :DP]
        B = AB[:, DP:]

        s, q, mn, mx, cntw = _sc_segment(A, B, T, src_s, attr_s, dst_s, offw)
        cntp = cntw[:, :1]

        Wpost = cp["post_nn"]["W"]
        blocks = [Wpost[i * d:(i + 1) * d] for i in range(6)]
        scaled = [blocks[0]] + [wagg[i] * blocks[i + 1] for i in range(5)]
        wstack = jnp.stack([_pad2(wb, DP, DP) for wb in scaled])
        hpost = _post(h, s, q, mn, mx, cntp, wstack,
                      _pad1(cp["post_nn"]["b"], DP))

        h = _mm(hpost, _pad2(cp["lin"]["W"], DP, DP),
                _pad1(cp["lin"]["b"], DP))

        stats = _bn_stats(h, n)
        mu = stats[0] / n
        var = stats[1] / n - mu * mu
        gamma = _pad1(cp["bn_gamma"], DP)
        beta = _pad1(cp["bn_beta"], DP)
        scale = gamma * jax.lax.rsqrt(var + 1e-5)
        shift = beta - mu * scale
        h = _bn_apply_relu(h, scale, shift)

    f = _mm(h, _pad2(params["fp1"]["W"], DP, 384),
            _pad1(params["fp1"]["b"], 384), act="relu")
    out = _mm(f, _pad2(params["fp2"]["W"], 384, 128),
              _pad1(params["fp2"]["b"], 128), bk=384)
    return out[:n, :3]
